# baseline (speedup 1.0000x reference)
import jax
import jax.numpy as jnp
from jax import lax
from jax.experimental import pallas as pl
from jax.experimental.pallas import tpu as pltpu
from jax.experimental.pallas import tpu_sc as plsc

KK = 256
N = 32768
ROWS = 1024
L = 16
NC = 2
NS = 16
NW = NC * NS
RPW = ROWS // NW
NBINS = 8192
NCOARSE = 512
CAP = 512
NVR = CAP // L
NEGB = 4096


def _shrl(x, s):
    return lax.shift_right_logical(x, jnp.full((L,), s, jnp.int32))


def _body(x_hbm, out_hbm, row_v, hist_v, csuf_v, cand_v, outb_v, sem, sem2):
    wid = lax.axis_index("s") * NC + lax.axis_index("c")
    base_row = wid * RPW

    iota = lax.iota(jnp.int32, L)
    zeros_i = jnp.zeros((L,), jnp.int32)
    ones_i = jnp.ones((L,), jnp.int32)
    kvec = jnp.full((L,), KK, jnp.int32)
    neg_inf = jnp.full((L,), -jnp.inf, jnp.float32)
    sh31 = jnp.full((L,), 31, jnp.int32)
    min_i32 = jnp.full((L,), -2147483648, jnp.int32)
    capm1 = jnp.full((L,), CAP - 1, jnp.int32)

    def fkey(v):
        bi = lax.bitcast_convert_type(v, jnp.int32)
        return bi ^ (lax.shift_right_arithmetic(bi, sh31) | min_i32)

    def bin_to_threshold(b):
        tk = lax.shift_left(b, jnp.full((L,), 19, jnp.int32))
        tb = tk ^ (
            jnp.bitwise_not(lax.shift_right_arithmetic(tk, sh31)) | min_i32
        )
        return lax.bitcast_convert_type(tb, jnp.float32)

    csuf_v[pl.ds(NCOARSE, L)] = zeros_i

    pltpu.async_copy(x_hbm.at[base_row], row_v.at[pl.ds(0, N)], sem)

    def do_row(r, bstar_prev):
        pltpu.make_async_copy(x_hbm.at[base_row], row_v.at[pl.ds(0, N)], sem).wait()

        @pl.when(r + 1 < RPW)
        def _():
            nxt = (r + 1) & 1
            pltpu.async_copy(
                x_hbm.at[base_row + r + 1], row_v.at[pl.ds(nxt * N, N)], sem
            )

        off = (r & 1) * N
        s19 = jnp.full((L,), 19, jnp.int32)
        tf_pred = lax.bitcast_convert_type(
            lax.shift_left(bstar_prev, s19), jnp.float32
        )
        negb = jnp.full((L,), NEGB, jnp.int32)

        csuf_v[pl.ds(NEGB // L, L)] = zeros_i
        @plsc.parallel_loop(0, (NEGB + L) // L, unroll=8)
        def z_hist(i):
            hist_v[pl.ds(i * L, L)] = zeros_i

        @plsc.parallel_loop(0, NVR, unroll=8)
        def z_cand(i):
            cand_v[pl.ds(i * L, L)] = neg_inf

        @plsc.parallel_loop(0, N // L, unroll=8, carry=zeros_i - 1)
        def fused(i, cnt):
            v = row_v[pl.ds(off + i * L, L)]
            bi = lax.bitcast_convert_type(v, jnp.int32)
            fb = jnp.minimum(_shrl(bi, 19), negb)
            plsc.addupdate_scatter(hist_v, [fb], ones_i)
            m = v >= tf_pred
            pos = plsc.cumsum(ones_i, mask=m)
            dest = jnp.minimum(cnt + pos, capm1)
            plsc.store_scatter(cand_v, [dest], v, mask=m)
            return cnt + plsc.all_reduce_population_count(m)

        cntm1 = fused

        def radix_threshold(jstart):
            def not_crossed(carry):
                j, csum = carry
                return (csum < KK) & (j >= 0)

            def scan_chunk(carry):
                j, csum = carry
                base = j * (L * L) + iota * L
                s = plsc.load_gather(hist_v, [base])
                for m in range(1, L):
                    s = s + plsc.load_gather(hist_v, [base + m])
                c = plsc.cumsum(lax.rev(s, (0,))) + csum
                csuf_v[pl.ds(j * L, L)] = lax.rev(c, (0,))
                return j - 1, jnp.max(c)

            jm1, csumf = lax.while_loop(
                not_crossed, scan_chunk, (jnp.int32(jstart), jnp.int32(0))
            )
            jlast = jm1 + 1
            s = csuf_v[pl.ds(jlast * L, L)]
            pcm = plsc.all_reduce_population_count(s >= kvec)
            cbin = jlast * L + pcm - 1
            above = plsc.load_gather(csuf_v, [cbin + 1])
            cb = jnp.max(jnp.maximum(cbin, zeros_i))

            hh = hist_v[pl.ds(cb * L, L)]
            c2 = plsc.cumsum(lax.rev(hh, (0,))) + above
            nm = plsc.all_reduce_population_count(jnp.logical_not(c2 >= kvec))
            return csumf, (cb * L + (L - 1)) - nm

        csum_hot, bstar = radix_threshold(NEGB // (L * L) - 1)
        crossed = csum_hot >= KK

        def hot_tf():
            return lax.bitcast_convert_type(
                lax.shift_left(bstar, s19), jnp.float32
            )

        def cold_tf():
            @plsc.parallel_loop(0, NBINS // L, unroll=8)
            def z_cold(i):
                hist_v[pl.ds(i * L, L)] = zeros_i

            @plsc.parallel_loop(0, N // L, unroll=8)
            def h_cold(i):
                v = row_v[pl.ds(off + i * L, L)]
                plsc.addupdate_scatter(hist_v, [_shrl(fkey(v), 19)], ones_i)

            _, bstar2 = radix_threshold(NCOARSE // L - 1)
            return bin_to_threshold(bstar2)

        tf_exact = lax.cond(crossed, hot_tf, cold_tf)

        good = (
            crossed
            & (jnp.max(bstar_prev) <= jnp.max(bstar))
            & (jnp.max(cntm1) < CAP)
        )

        @pl.when(jnp.logical_not(good))
        def _():
            @plsc.parallel_loop(0, NVR, unroll=8)
            def z_cand2(i):
                cand_v[pl.ds(i * L, L)] = neg_inf

            @plsc.parallel_loop(0, N // L, unroll=8, carry=zeros_i - 1)
            def collect(i, cnt):
                v = row_v[pl.ds(off + i * L, L)]
                m = v >= tf_exact
                pos = plsc.cumsum(ones_i, mask=m)
                dest = jnp.minimum(cnt + pos, capm1)
                plsc.store_scatter(cand_v, [dest], v, mask=m)
                return cnt + plsc.all_reduce_population_count(m)

        V = [cand_v[pl.ds(v * L, L)] for v in range(NVR)]
        for v in range(NVR):
            V[v] = plsc.sort_key_val(V[v], V[v], descending=(v & 1) == 0)[0]
        for kv in (2, 4, 8, 16, 32):
            jv = kv // 2
            while jv >= 1:
                for v in range(NVR):
                    p = v ^ jv
                    if p > v:
                        hi = jnp.maximum(V[v], V[p])
                        lo = jnp.minimum(V[v], V[p])
                        if (v & kv) == 0:
                            V[v], V[p] = hi, lo
                        else:
                            V[v], V[p] = lo, hi
                jv //= 2
            for v in range(NVR):
                V[v] = plsc.sort_key_val(V[v], V[v], descending=(v & kv) == 0)[0]

        @pl.when(r >= 2)
        def _():
            pltpu.make_async_copy(
                out_hbm.at[base_row], outb_v.at[pl.ds(0, KK)], sem2
            ).wait()

        ob = (r & 1) * KK
        for v in range(KK // L):
            outb_v[pl.ds(ob + v * L, L)] = V[v]
        pltpu.async_copy(
            outb_v.at[pl.ds(ob, KK)], out_hbm.at[base_row + r], sem2
        )
        return jnp.where(crossed, bstar, jnp.full((L,), NEGB - 1, jnp.int32))

    lax.fori_loop(0, RPW, do_row, jnp.full((L,), NEGB - 1, jnp.int32))
    for _ in range(2):
        pltpu.make_async_copy(
            out_hbm.at[base_row], outb_v.at[pl.ds(0, KK)], sem2
        ).wait()


def kernel(x):
    xf = x.reshape(ROWS, N)
    mesh = plsc.VectorSubcoreMesh(core_axis_name="c", subcore_axis_name="s")
    out = pl.kernel(
        _body,
        out_type=jax.ShapeDtypeStruct((ROWS, KK), jnp.float32),
        mesh=mesh,
        compiler_params=pltpu.CompilerParams(needs_layout_passes=False),
        scratch_types=[
            pltpu.VMEM((2 * N,), jnp.float32),
            pltpu.VMEM((NBINS,), jnp.int32),
            pltpu.VMEM((NCOARSE + L,), jnp.int32),
            pltpu.VMEM((CAP,), jnp.float32),
            pltpu.VMEM((2 * KK,), jnp.float32),
            pltpu.SemaphoreType.DMA,
            pltpu.SemaphoreType.DMA,
        ],
    )(xf)
    return out.reshape(64, 16, KK)


# speedup vs baseline: 2.0927x; 2.0927x over previous
import jax
import jax.numpy as jnp
from jax import lax
from jax.experimental import pallas as pl
from jax.experimental.pallas import tpu as pltpu
from jax.experimental.pallas import tpu_sc as plsc

KK = 256
N = 32768
ROWS = 1024
L = 16
NC = 2
NS = 16
NW = NC * NS
RPW = ROWS // NW
NBINS = 8192
NCOARSE = 512
CAP = 512
NVR = CAP // L


def _shrl(x, s):
    return lax.shift_right_logical(x, jnp.full((L,), s, jnp.int32))


def _body(x_hbm, out_hbm, row_v, hist_v, csuf_v, cand_v, outb_v, sem, sem2):
    wid = lax.axis_index("s") * NC + lax.axis_index("c")
    base_row = wid * RPW

    iota = lax.iota(jnp.int32, L)
    zeros_i = jnp.zeros((L,), jnp.int32)
    ones_i = jnp.ones((L,), jnp.int32)
    kvec = jnp.full((L,), KK, jnp.int32)
    neg_inf = jnp.full((L,), -jnp.inf, jnp.float32)
    sh31 = jnp.full((L,), 31, jnp.int32)
    min_i32 = jnp.full((L,), -2147483648, jnp.int32)
    capm1 = jnp.full((L,), CAP - 1, jnp.int32)

    def fkey(v):
        bi = lax.bitcast_convert_type(v, jnp.int32)
        return bi ^ (lax.shift_right_arithmetic(bi, sh31) | min_i32)

    def bin_to_threshold(b):
        tk = lax.shift_left(b, jnp.full((L,), 19, jnp.int32))
        tb = tk ^ (
            jnp.bitwise_not(lax.shift_right_arithmetic(tk, sh31)) | min_i32
        )
        return lax.bitcast_convert_type(tb, jnp.float32)

    csuf_v[pl.ds(NCOARSE, L)] = zeros_i

    pltpu.async_copy(x_hbm.at[base_row], row_v.at[pl.ds(0, N)], sem)

    def do_row(r, bstar_prev):
        pltpu.make_async_copy(x_hbm.at[base_row], row_v.at[pl.ds(0, N)], sem).wait()

        @pl.when(r + 1 < RPW)
        def _():
            nxt = (r + 1) & 1
            pltpu.async_copy(
                x_hbm.at[base_row + r + 1], row_v.at[pl.ds(nxt * N, N)], sem
            )

        off = (r & 1) * N
        tf_pred = bin_to_threshold(bstar_prev)

        @plsc.parallel_loop(0, NBINS // L, unroll=8)
        def z_hist(i):
            hist_v[pl.ds(i * L, L)] = zeros_i

        @plsc.parallel_loop(0, NVR, unroll=8)
        def z_cand(i):
            cand_v[pl.ds(i * L, L)] = neg_inf

        @plsc.parallel_loop(0, N // L, unroll=8, carry=zeros_i - 1)
        def fused(i, cnt):
            v = row_v[pl.ds(off + i * L, L)]
            fb = _shrl(fkey(v), 19)
            plsc.addupdate_scatter(hist_v, [fb], ones_i)
            m = v >= tf_pred
            pos = plsc.cumsum(ones_i, mask=m)
            dest = jnp.minimum(cnt + pos, capm1)
            plsc.store_scatter(cand_v, [dest], v, mask=m)
            return cnt + plsc.all_reduce_population_count(m)

        cntm1 = fused

        def not_crossed(carry):
            j, csum = carry
            return (csum < KK) & (j >= 0)

        def scan_chunk(carry):
            j, csum = carry
            base = j * (L * L) + iota * L
            s = plsc.load_gather(hist_v, [base])
            for m in range(1, L):
                s = s + plsc.load_gather(hist_v, [base + m])
            c = plsc.cumsum(lax.rev(s, (0,))) + csum
            csuf_v[pl.ds(j * L, L)] = lax.rev(c, (0,))
            return j - 1, jnp.max(c)

        jm1, _ = lax.while_loop(
            not_crossed, scan_chunk, (jnp.int32(NCOARSE // L - 1), jnp.int32(0))
        )
        jlast = jm1 + 1
        s = csuf_v[pl.ds(jlast * L, L)]
        pcm = plsc.all_reduce_population_count(s >= kvec)
        cbin = jlast * L + pcm - 1
        above = plsc.load_gather(csuf_v, [cbin + 1])
        cb = jnp.max(cbin)

        hh = hist_v[pl.ds(cb * L, L)]
        c2 = plsc.cumsum(lax.rev(hh, (0,))) + above
        nm = plsc.all_reduce_population_count(jnp.logical_not(c2 >= kvec))
        bstar = (cb * L + (L - 1)) - nm

        good = (jnp.max(bstar_prev) <= jnp.max(bstar)) & (
            jnp.max(cntm1) < CAP
        )

        @pl.when(jnp.logical_not(good))
        def _():
            tf = bin_to_threshold(bstar)

            @plsc.parallel_loop(0, NVR, unroll=8)
            def z_cand2(i):
                cand_v[pl.ds(i * L, L)] = neg_inf

            @plsc.parallel_loop(0, N // L, unroll=8, carry=zeros_i - 1)
            def collect(i, cnt):
                v = row_v[pl.ds(off + i * L, L)]
                m = v >= tf
                pos = plsc.cumsum(ones_i, mask=m)
                dest = jnp.minimum(cnt + pos, capm1)
                plsc.store_scatter(cand_v, [dest], v, mask=m)
                return cnt + plsc.all_reduce_population_count(m)

        V = [cand_v[pl.ds(v * L, L)] for v in range(NVR)]
        for v in range(NVR):
            V[v] = plsc.sort_key_val(V[v], V[v], descending=(v & 1) == 0)[0]
        for kv in (2, 4, 8, 16, 32):
            jv = kv // 2
            while jv >= 1:
                for v in range(NVR):
                    p = v ^ jv
                    if p > v:
                        hi = jnp.maximum(V[v], V[p])
                        lo = jnp.minimum(V[v], V[p])
                        if (v & kv) == 0:
                            V[v], V[p] = hi, lo
                        else:
                            V[v], V[p] = lo, hi
                jv //= 2
            for v in range(NVR):
                V[v] = plsc.sort_key_val(V[v], V[v], descending=(v & kv) == 0)[0]

        @pl.when(r >= 2)
        def _():
            pltpu.make_async_copy(
                out_hbm.at[base_row], outb_v.at[pl.ds(0, KK)], sem2
            ).wait()

        ob = (r & 1) * KK
        for v in range(KK // L):
            outb_v[pl.ds(ob + v * L, L)] = V[v]
        pltpu.async_copy(
            outb_v.at[pl.ds(ob, KK)], out_hbm.at[base_row + r], sem2
        )
        return bstar

    lax.fori_loop(0, RPW, do_row, jnp.full((L,), NBINS - 1, jnp.int32))
    for _ in range(2):
        pltpu.make_async_copy(
            out_hbm.at[base_row], outb_v.at[pl.ds(0, KK)], sem2
        ).wait()


def kernel(x):
    xf = x.reshape(ROWS, N)
    mesh = plsc.VectorSubcoreMesh(core_axis_name="c", subcore_axis_name="s")
    out = pl.kernel(
        _body,
        out_type=jax.ShapeDtypeStruct((ROWS, KK), jnp.float32),
        mesh=mesh,
        compiler_params=pltpu.CompilerParams(needs_layout_passes=False),
        scratch_types=[
            pltpu.VMEM((2 * N,), jnp.float32),
            pltpu.VMEM((NBINS,), jnp.int32),
            pltpu.VMEM((NCOARSE + L,), jnp.int32),
            pltpu.VMEM((CAP,), jnp.float32),
            pltpu.VMEM((2 * KK,), jnp.float32),
            pltpu.SemaphoreType.DMA,
            pltpu.SemaphoreType.DMA,
        ],
    )(xf)
    return out.reshape(64, 16, KK)
